# baseline (speedup 1.0000x reference)
import jax
import jax.numpy as jnp
from jax.experimental import pallas as pl

E = 160000
CTX = 5
HID = 128
OUT_DIM = 40
N_TYPES = 15
KIN = 24
BE = 16000


def _mlp_body(x_ref, wcat_ref, w2_ref, b2_ref, w3_ref, b3_ref, o_ref):
    x = x_ref[...]
    z1 = jnp.dot(x, wcat_ref[...].astype(jnp.bfloat16),
                 preferred_element_type=jnp.float32).astype(jnp.bfloat16)
    h1 = z1 * jax.nn.sigmoid(z1)
    z2 = (jnp.dot(h1, w2_ref[...], preferred_element_type=jnp.float32)
          ).astype(jnp.bfloat16) + b2_ref[...]
    h2 = z2 * jax.nn.sigmoid(z2)
    o_ref[:, 0, :] = jnp.dot(h2, w3_ref[...],
                             preferred_element_type=jnp.float32) + b3_ref[...]


def kernel(edge_attr, edge_type, layer_embed_w, W_proj, b_proj,
           W1, b1, W2, b2, W3, b3, k=1):
    e = edge_attr.shape[0]
    wc = jnp.dot(W1, W_proj)
    bc = (jnp.dot(W1, b_proj) + b1).reshape(1, HID)
    t2 = jnp.dot(layer_embed_w, W1.T) + bc
    wcat = jnp.zeros((KIN, HID), jnp.float32)
    wcat = wcat.at[:CTX, :].set(wc.T)
    wcat = wcat.at[CTX:CTX + N_TYPES, :].set(t2)
    w2_t = W2.T.astype(jnp.bfloat16)
    b2r = b2.reshape(1, HID).astype(jnp.bfloat16)
    w3_t = W3.T.astype(jnp.bfloat16)
    b3r = b3.reshape(1, OUT_DIM)
    onehot = (edge_type.astype(jnp.int32)[:, None]
              == jnp.arange(N_TYPES, dtype=jnp.int32)[None, :])
    x = jnp.concatenate(
        [edge_attr.astype(jnp.bfloat16),
         onehot.astype(jnp.bfloat16),
         jnp.zeros((e, KIN - CTX - N_TYPES), jnp.bfloat16)], axis=1)

    grid = (e // BE,)
    rep = lambda i: (0, 0)
    out = pl.pallas_call(
        _mlp_body,
        grid=grid,
        in_specs=[
            pl.BlockSpec((BE, KIN), lambda i: (i, 0)),
            pl.BlockSpec((KIN, HID), rep),
            pl.BlockSpec((HID, HID), rep),
            pl.BlockSpec((1, HID), rep),
            pl.BlockSpec((HID, OUT_DIM), rep),
            pl.BlockSpec((1, OUT_DIM), rep),
        ],
        out_specs=pl.BlockSpec((BE, 1, OUT_DIM), lambda i: (i, 0, 0)),
        out_shape=jax.ShapeDtypeStruct((e, 1, OUT_DIM), jnp.float32),
    )(x, wcat, w2_t, b2r, w3_t, b3r)
    return out


# speedup vs baseline: 1.6847x; 1.6847x over previous
import jax
import jax.numpy as jnp
from jax.experimental import pallas as pl

E = 160000
CTX = 5
HID = 128
OUT_DIM = 40
N_TYPES = 15
BE = 16384
ROWS = BE // 128


def _mlp_body(ea_ref, et_ref, wc_ref, t2_ref, w2_ref, b2_ref,
              w3_ref, b3_ref, o_ref):
    ea = ea_ref[...].astype(jnp.bfloat16)
    etl = et_ref[...]
    oh3 = (etl[:, :, None] == jax.lax.broadcasted_iota(jnp.int32, (1, 1, 16), 2))
    oh = oh3.reshape(BE, 16).astype(jnp.bfloat16)
    z1 = (jnp.dot(ea, wc_ref[...].astype(jnp.bfloat16),
                  preferred_element_type=jnp.float32)
          + jnp.dot(oh, t2_ref[...], preferred_element_type=jnp.float32)
          ).astype(jnp.bfloat16)
    h1 = z1 * jax.nn.sigmoid(z1)
    z2 = (jnp.dot(h1, w2_ref[...], preferred_element_type=jnp.float32)
          ).astype(jnp.bfloat16) + b2_ref[...]
    h2 = z2 * jax.nn.sigmoid(z2)
    o_ref[...] = jnp.dot(h2, w3_ref[...],
                         preferred_element_type=jnp.float32) + b3_ref[...]


def kernel(edge_attr, edge_type, layer_embed_w, W_proj, b_proj,
           W1, b1, W2, b2, W3, b3, k=1):
    e = edge_attr.shape[0]
    wc_t = jnp.dot(W1, W_proj).T
    bc = (jnp.dot(W1, b_proj) + b1).reshape(1, HID)
    t2 = jnp.dot(layer_embed_w, W1.T) + bc
    t2p = jnp.zeros((16, HID), jnp.bfloat16).at[:N_TYPES, :].set(
        t2.astype(jnp.bfloat16))
    w2_t = W2.T.astype(jnp.bfloat16)
    b2r = b2.reshape(1, HID).astype(jnp.bfloat16)
    w3_t = W3.T.astype(jnp.bfloat16)
    b3r = b3.reshape(1, OUT_DIM)
    et2 = edge_type.astype(jnp.int32).reshape(e // 128, 128)

    grid = (pl.cdiv(e, BE),)
    rep = lambda i: (0, 0)
    out = pl.pallas_call(
        _mlp_body,
        grid=grid,
        in_specs=[
            pl.BlockSpec((BE, CTX), lambda i: (i, 0)),
            pl.BlockSpec((ROWS, 128), lambda i: (i, 0)),
            pl.BlockSpec((CTX, HID), rep),
            pl.BlockSpec((16, HID), rep),
            pl.BlockSpec((HID, HID), rep),
            pl.BlockSpec((1, HID), rep),
            pl.BlockSpec((HID, OUT_DIM), rep),
            pl.BlockSpec((1, OUT_DIM), rep),
        ],
        out_specs=pl.BlockSpec((BE, OUT_DIM), lambda i: (i, 0)),
        out_shape=jax.ShapeDtypeStruct((e, OUT_DIM), jnp.float32),
    )(edge_attr, et2, wc_t, t2p, w2_t, b2r, w3_t, b3r)
    return out.reshape(e, 1, OUT_DIM)


# speedup vs baseline: 1.8640x; 1.1064x over previous
import jax
import jax.numpy as jnp
from jax.experimental import pallas as pl

E = 160000
CTX = 5
HID = 128
OUT_DIM = 40
N_TYPES = 15
KIN = 24
BE = 16000


def _mlp_body(x_ref, wcat_ref, w2_ref, b2_ref, w3_ref, b3_ref, o_ref):
    x = x_ref[...]
    z1 = jnp.dot(x, wcat_ref[...].astype(jnp.bfloat16),
                 preferred_element_type=jnp.float32).astype(jnp.bfloat16)
    h1 = z1 * jax.nn.sigmoid(z1)
    z2 = (jnp.dot(h1, w2_ref[...], preferred_element_type=jnp.float32)
          ).astype(jnp.bfloat16) + b2_ref[...]
    h2 = z2 * jax.nn.sigmoid(z2)
    o_ref[...] = jnp.dot(h2, w3_ref[...],
                         preferred_element_type=jnp.float32) + b3_ref[...]


def kernel(edge_attr, edge_type, layer_embed_w, W_proj, b_proj,
           W1, b1, W2, b2, W3, b3, k=1):
    e = edge_attr.shape[0]
    wc = jnp.dot(W1, W_proj)
    bc = (jnp.dot(W1, b_proj) + b1).reshape(1, HID)
    t2 = jnp.dot(layer_embed_w, W1.T) + bc
    wcat = jnp.zeros((KIN, HID), jnp.float32)
    wcat = wcat.at[:CTX, :].set(wc.T)
    wcat = wcat.at[CTX:CTX + N_TYPES, :].set(t2)
    w2_t = W2.T.astype(jnp.bfloat16)
    b2r = b2.reshape(1, HID).astype(jnp.bfloat16)
    w3_t = W3.T.astype(jnp.bfloat16)
    b3r = b3.reshape(1, OUT_DIM)
    onehot = (edge_type.astype(jnp.int32)[:, None]
              == jnp.arange(N_TYPES, dtype=jnp.int32)[None, :])
    x = jnp.concatenate(
        [edge_attr.astype(jnp.bfloat16),
         onehot.astype(jnp.bfloat16),
         jnp.zeros((e, KIN - CTX - N_TYPES), jnp.bfloat16)], axis=1)

    grid = (e // BE,)
    rep = lambda i: (0, 0)
    out = pl.pallas_call(
        _mlp_body,
        grid=grid,
        in_specs=[
            pl.BlockSpec((BE, KIN), lambda i: (i, 0)),
            pl.BlockSpec((KIN, HID), rep),
            pl.BlockSpec((HID, HID), rep),
            pl.BlockSpec((1, HID), rep),
            pl.BlockSpec((HID, OUT_DIM), rep),
            pl.BlockSpec((1, OUT_DIM), rep),
        ],
        out_specs=pl.BlockSpec((BE, OUT_DIM), lambda i: (i, 0)),
        out_shape=jax.ShapeDtypeStruct((e, OUT_DIM), jnp.float32),
    )(x, wcat, w2_t, b2r, w3_t, b3r)
    return out.reshape(e, 1, OUT_DIM)
